# baseline (speedup 1.0000x reference)
import functools

import jax
import jax.numpy as jnp
from jax import lax
from jax.experimental import pallas as pl
from jax.experimental.pallas import tpu as pltpu
from jax.experimental.pallas import tpu_sc as plsc

_S = 400.0
_L = 16


@functools.cache
def _make_kernel(V, D, B):
    NC, NS = 2, 16
    NW = NC * NS
    assert B % (8 * NW) == 0 and D % _L == 0
    b_per_w = B // NW
    mesh = plsc.VectorSubcoreMesh(core_axis_name="c", subcore_axis_name="s")

    NCH = 8
    CB = b_per_w // NCH

    @functools.partial(
        pl.kernel,
        mesh=mesh,
        out_type=jax.ShapeDtypeStruct((B, D), jnp.float32),
        scratch_types=[
            pltpu.VMEM((b_per_w,), jnp.int32),
            pltpu.VMEM((b_per_w, D), jnp.float32),
            pltpu.SemaphoreType.DMA((NCH,)),
            pltpu.SemaphoreType.DMA((NCH,)),
        ],
    )
    def gather_sigmoid(idx_hbm, table_hbm, out_hbm, idx_v, rows_v, gsem, wsem):
        wid = lax.axis_index("s") * NC + lax.axis_index("c")
        base = wid * b_per_w
        pltpu.sync_copy(idx_hbm.at[pl.ds(base, b_per_w)], idx_v)

        gathers = [
            pltpu.async_copy(
                table_hbm.at[idx_v.at[pl.ds(c * CB, CB)]],
                rows_v.at[pl.ds(c * CB, CB)],
                gsem.at[c],
            )
            for c in range(NCH)
        ]

        writes = []
        for c in range(NCH):
            gathers[c].wait()

            @plsc.parallel_loop(c * CB, (c + 1) * CB, step=1, unroll=4)
            def _(r):
                for j in range(D // _L):
                    h = rows_v[r, pl.ds(j * _L, _L)]
                    z = jnp.exp(h * -_S)
                    rows_v[r, pl.ds(j * _L, _L)] = 1.0 / (1.0 + z)
            writes.append(
                pltpu.async_copy(
                    rows_v.at[pl.ds(c * CB, CB)],
                    out_hbm.at[pl.ds(base + c * CB, CB)],
                    wsem.at[c],
                )
            )
        for w in writes:
            w.wait()

    return gather_sigmoid


def kernel(t, table):
    (B,) = t.shape
    V, D = table.shape
    return _make_kernel(V, D, B)(t.astype(jnp.int32), table)


# speedup vs baseline: 1.0436x; 1.0436x over previous
import functools

import jax
import jax.numpy as jnp
from jax import lax
from jax.experimental import pallas as pl
from jax.experimental.pallas import tpu as pltpu
from jax.experimental.pallas import tpu_sc as plsc

_S = 400.0
_L = 16


@functools.cache
def _make_kernel(V, D, B):
    NC, NS = 2, 16
    NW = NC * NS
    assert B % (8 * NW) == 0 and D % _L == 0
    b_per_w = B // NW
    mesh = plsc.VectorSubcoreMesh(core_axis_name="c", subcore_axis_name="s")

    NCH = 8
    CB = b_per_w // NCH

    @functools.partial(
        pl.kernel,
        mesh=mesh,
        out_type=jax.ShapeDtypeStruct((B, D), jnp.float32),
        scratch_types=[
            pltpu.VMEM((b_per_w,), jnp.int32),
            pltpu.VMEM((b_per_w, D), jnp.float32),
            pltpu.SemaphoreType.DMA((NCH,)),
            pltpu.SemaphoreType.DMA((NCH,)),
        ],
    )
    def gather_sigmoid(idx_hbm, table_hbm, out_hbm, idx_v, rows_v, gsem, wsem):
        wid = lax.axis_index("s") * NC + lax.axis_index("c")
        base = wid * b_per_w
        pltpu.sync_copy(idx_hbm.at[pl.ds(base, b_per_w)], idx_v)

        gathers = [
            pltpu.async_copy(
                table_hbm.at[idx_v.at[pl.ds(c * CB, CB)]],
                rows_v.at[pl.ds(c * CB, CB)],
                gsem.at[c],
            )
            for c in range(NCH)
        ]

        writes = []
        for c in range(NCH):
            gathers[c].wait()

            @plsc.parallel_loop(c * CB, (c + 1) * CB, step=1, unroll=2)
            def _(r):
                for j in range(D // _L):
                    h = rows_v[r, pl.ds(j * _L, _L)]
                    z = jnp.exp(h * -_S)
                    rows_v[r, pl.ds(j * _L, _L)] = 1.0 / (1.0 + z)
            writes.append(
                pltpu.async_copy(
                    rows_v.at[pl.ds(c * CB, CB)],
                    out_hbm.at[pl.ds(base + c * CB, CB)],
                    wsem.at[c],
                )
            )
        for w in writes:
            w.wait()

    return gather_sigmoid


def kernel(t, table):
    (B,) = t.shape
    V, D = table.shape
    return _make_kernel(V, D, B)(t.astype(jnp.int32), table)


# speedup vs baseline: 1.0570x; 1.0129x over previous
import functools

import jax
import jax.numpy as jnp
from jax import lax
from jax.experimental import pallas as pl
from jax.experimental.pallas import tpu as pltpu
from jax.experimental.pallas import tpu_sc as plsc

_S = 400.0
_L = 16


@functools.cache
def _make_kernel(V, D, B):
    NC, NS = 2, 16
    NW = NC * NS
    assert B % (8 * NW) == 0 and D % _L == 0
    b_per_w = B // NW
    mesh = plsc.VectorSubcoreMesh(core_axis_name="c", subcore_axis_name="s")

    NCH = 4
    CB = b_per_w // NCH

    @functools.partial(
        pl.kernel,
        mesh=mesh,
        out_type=jax.ShapeDtypeStruct((B, D), jnp.float32),
        scratch_types=[
            pltpu.VMEM((b_per_w,), jnp.int32),
            pltpu.VMEM((b_per_w, D), jnp.float32),
            pltpu.SemaphoreType.DMA((NCH,)),
            pltpu.SemaphoreType.DMA((NCH,)),
        ],
    )
    def gather_sigmoid(idx_hbm, table_hbm, out_hbm, idx_v, rows_v, gsem, wsem):
        wid = lax.axis_index("s") * NC + lax.axis_index("c")
        base = wid * b_per_w
        pltpu.sync_copy(idx_hbm.at[pl.ds(base, b_per_w)], idx_v)

        gathers = [
            pltpu.async_copy(
                table_hbm.at[idx_v.at[pl.ds(c * CB, CB)]],
                rows_v.at[pl.ds(c * CB, CB)],
                gsem.at[c],
            )
            for c in range(NCH)
        ]

        writes = []
        for c in range(NCH):
            gathers[c].wait()

            @plsc.parallel_loop(c * CB, (c + 1) * CB, step=1, unroll=2)
            def _(r):
                for j in range(D // _L):
                    h = rows_v[r, pl.ds(j * _L, _L)]
                    z = jnp.exp(h * -_S)
                    rows_v[r, pl.ds(j * _L, _L)] = 1.0 / (1.0 + z)
            writes.append(
                pltpu.async_copy(
                    rows_v.at[pl.ds(c * CB, CB)],
                    out_hbm.at[pl.ds(base + c * CB, CB)],
                    wsem.at[c],
                )
            )
        for w in writes:
            w.wait()

    return gather_sigmoid


def kernel(t, table):
    (B,) = t.shape
    V, D = table.shape
    return _make_kernel(V, D, B)(t.astype(jnp.int32), table)
